# baseline (speedup 1.0000x reference)
import functools

import jax
import jax.numpy as jnp
from jax import lax
from jax.experimental import pallas as pl
from jax.experimental.pallas import tpu as pltpu
from jax.experimental.pallas import tpu_sc as plsc

_D = 64
_GRP = 128
_K = 5
_CHUNK = _K * _GRP


@functools.lru_cache(maxsize=None)
def _build(B, V):
    info = plsc.get_sparse_core_info()
    nw = info.num_cores * info.num_subcores
    b_per_w = B // nw
    nchunks = b_per_w // _CHUNK
    assert b_per_w % _CHUNK == 0 and nchunks % 2 == 0

    mesh = plsc.VectorSubcoreMesh(core_axis_name="c", subcore_axis_name="s")

    @functools.partial(
        pl.kernel,
        out_type=jax.ShapeDtypeStruct((B, _D), jnp.float32),
        mesh=mesh,
        compiler_params=pltpu.CompilerParams(use_tc_tiling_on_sc=False),
        scratch_types=[
            pltpu.VMEM((_K, _GRP), jnp.int32),
            pltpu.VMEM((_K, _GRP), jnp.int32),
            pltpu.VMEM((_CHUNK, _D), jnp.float32),
            pltpu.VMEM((_CHUNK, _D), jnp.float32),
            pltpu.SemaphoreType.DMA,
            pltpu.SemaphoreType.DMA,
        ],
    )
    def emb_lookup(x_hbm, emb_hbm, out_hbm, idx0, idx1, rows0, rows1,
                   sem0, sem1):
        wid = lax.axis_index("s") * info.num_cores + lax.axis_index("c")
        base = wid * b_per_w
        cbase = wid * nchunks

        idx_s = (idx0, idx1)
        rows_s = (rows0, rows1)
        sems = (sem0, sem1)

        def start(g, slot):
            pltpu.sync_copy(x_hbm.at[cbase + g], idx_s[slot])
            for j in range(_K):
                pltpu.async_copy(
                    emb_hbm.at[idx_s[slot].at[j]],
                    rows_s[slot].at[pl.ds(j * _GRP, _GRP)],
                    sems[slot],
                )

        def finish(g, slot):
            pltpu.make_async_copy(
                emb_hbm.at[pl.ds(0, _CHUNK)], rows_s[slot], sems[slot]
            ).wait()
            pltpu.sync_copy(
                rows_s[slot], out_hbm.at[pl.ds(base + g * _CHUNK, _CHUNK)]
            )

        start(0, 0)

        @pl.loop(0, nchunks, step=2)
        def _pair(g):
            start(g + 1, 1)
            finish(g, 0)

            @pl.when(g + 2 < nchunks)
            def _():
                start(g + 2, 0)

            finish(g + 1, 1)

    return emb_lookup


def kernel(x, emb):
    b, h = x.shape
    v, d = emb.shape
    flat_idx = x.reshape(-1).astype(jnp.int32).reshape(-1, _K, _GRP)
    out = _build(b * h, v)(flat_idx, emb)
    return out.reshape(b, h, d)


# speedup vs baseline: 1.0128x; 1.0128x over previous
import functools

import jax
import jax.numpy as jnp
from jax import lax
from jax.experimental import pallas as pl
from jax.experimental.pallas import tpu as pltpu
from jax.experimental.pallas import tpu_sc as plsc

_D = 64
_GRP = 128
_K = 5
_CHUNK = _K * _GRP
_NSLOT = 3


@functools.lru_cache(maxsize=None)
def _build(B, V):
    info = plsc.get_sparse_core_info()
    nw = info.num_cores * info.num_subcores
    b_per_w = B // nw
    nchunks = b_per_w // _CHUNK
    assert b_per_w % _CHUNK == 0

    mesh = plsc.VectorSubcoreMesh(core_axis_name="c", subcore_axis_name="s")

    @functools.partial(
        pl.kernel,
        out_type=jax.ShapeDtypeStruct((B, _D), jnp.float32),
        mesh=mesh,
        compiler_params=pltpu.CompilerParams(use_tc_tiling_on_sc=False),
        scratch_types=[
            [pltpu.VMEM((_K, _GRP), jnp.int32)] * _NSLOT,
            [pltpu.VMEM((_CHUNK, _D), jnp.float32)] * _NSLOT,
            [pltpu.SemaphoreType.DMA] * _NSLOT,
            [pltpu.SemaphoreType.DMA] * _NSLOT,
        ],
    )
    def emb_lookup(x_hbm, emb_hbm, out_hbm, idx_s, rows_s, gsems, wsems):
        wid = lax.axis_index("s") * info.num_cores + lax.axis_index("c")
        base = wid * b_per_w
        cbase = wid * nchunks

        def start(g, slot):
            pltpu.sync_copy(x_hbm.at[cbase + g], idx_s[slot])
            for j in range(_K):
                pltpu.async_copy(
                    emb_hbm.at[idx_s[slot].at[j]],
                    rows_s[slot].at[pl.ds(j * _GRP, _GRP)],
                    gsems[slot],
                )

        def finish(g, slot):
            pltpu.make_async_copy(
                emb_hbm.at[pl.ds(0, _CHUNK)], rows_s[slot], gsems[slot]
            ).wait()
            pltpu.async_copy(
                rows_s[slot], out_hbm.at[pl.ds(base + g * _CHUNK, _CHUNK)],
                wsems[slot],
            )

        def wait_wb(slot):
            pltpu.make_async_copy(
                rows_s[slot], out_hbm.at[pl.ds(base, _CHUNK)], wsems[slot]
            ).wait()

        start(0, 0)

        @pl.loop(0, nchunks, step=_NSLOT)
        def _triplet(g):
            @pl.when(jnp.logical_and(g > 0, g + 1 < nchunks))
            def _():
                wait_wb(1)

            @pl.when(g + 1 < nchunks)
            def _():
                start(g + 1, 1)

            @pl.when(jnp.logical_and(g > 0, g + 2 < nchunks))
            def _():
                wait_wb(2)

            @pl.when(g + 2 < nchunks)
            def _():
                start(g + 2, 2)

            finish(g, 0)

            @pl.when(g + 1 < nchunks)
            def _():
                finish(g + 1, 1)

            @pl.when(g + 2 < nchunks)
            def _():
                finish(g + 2, 2)

            @pl.when(g + _NSLOT < nchunks)
            def _():
                wait_wb(0)
                start(g + _NSLOT, 0)

        for s in range(_NSLOT):
            wait_wb(s)

    return emb_lookup


def kernel(x, emb):
    b, h = x.shape
    v, d = emb.shape
    flat_idx = x.reshape(-1).astype(jnp.int32).reshape(-1, _K, _GRP)
    out = _build(b * h, v)(flat_idx, emb)
    return out.reshape(b, h, d)


# speedup vs baseline: 1.0146x; 1.0018x over previous
import functools

import jax
import jax.numpy as jnp
from jax import lax
from jax.experimental import pallas as pl
from jax.experimental.pallas import tpu as pltpu
from jax.experimental.pallas import tpu_sc as plsc

_D = 64
_H = 200
_R = 2
_NSLOT = 3
_SPLITS = ((0, 128), (128, 72))


@functools.lru_cache(maxsize=None)
def _build(B, V):
    info = plsc.get_sparse_core_info()
    nw = info.num_cores * info.num_subcores
    rows_per_w = B // nw
    nchunks = rows_per_w // _R
    assert B % nw == 0 and rows_per_w % _R == 0

    mesh = plsc.VectorSubcoreMesh(core_axis_name="c", subcore_axis_name="s")

    @functools.partial(
        pl.kernel,
        out_type=jax.ShapeDtypeStruct((B, _H, _D), jnp.float32),
        mesh=mesh,
        compiler_params=pltpu.CompilerParams(use_tc_tiling_on_sc=False),
        scratch_types=[
            [pltpu.VMEM((_R, _H), jnp.int32)] * _NSLOT,
            [pltpu.VMEM((_R, _H, _D), jnp.float32)] * _NSLOT,
            [pltpu.SemaphoreType.DMA] * _NSLOT,
            [pltpu.SemaphoreType.DMA] * _NSLOT,
        ],
    )
    def emb_lookup(x_hbm, emb_hbm, out_hbm, idx_s, rows_s, gsems, wsems):
        wid = lax.axis_index("s") * info.num_cores + lax.axis_index("c")
        base = wid * rows_per_w

        def start(g, slot):
            pltpu.sync_copy(x_hbm.at[pl.ds(base + g * _R, _R)], idx_s[slot])
            for r in range(_R):
                for off, ln in _SPLITS:
                    pltpu.async_copy(
                        emb_hbm.at[idx_s[slot].at[r, pl.ds(off, ln)]],
                        rows_s[slot].at[r, pl.ds(off, ln)],
                        gsems[slot],
                    )

        def finish(g, slot):
            pltpu.make_async_copy(
                out_hbm.at[pl.ds(0, _R)], rows_s[slot], gsems[slot]
            ).wait()
            pltpu.async_copy(
                rows_s[slot], out_hbm.at[pl.ds(base + g * _R, _R)],
                wsems[slot],
            )

        def wait_wb(slot):
            pltpu.make_async_copy(
                rows_s[slot], out_hbm.at[pl.ds(0, _R)], wsems[slot]
            ).wait()

        start(0, 0)

        @pl.loop(0, nchunks, step=_NSLOT)
        def _triplet(g):
            @pl.when(jnp.logical_and(g > 0, g + 1 < nchunks))
            def _():
                wait_wb(1)

            @pl.when(g + 1 < nchunks)
            def _():
                start(g + 1, 1)

            @pl.when(jnp.logical_and(g > 0, g + 2 < nchunks))
            def _():
                wait_wb(2)

            @pl.when(g + 2 < nchunks)
            def _():
                start(g + 2, 2)

            finish(g, 0)

            @pl.when(g + 1 < nchunks)
            def _():
                finish(g + 1, 1)

            @pl.when(g + 2 < nchunks)
            def _():
                finish(g + 2, 2)

            @pl.when(g + _NSLOT < nchunks)
            def _():
                wait_wb(0)
                start(g + _NSLOT, 0)

        for s in range(_NSLOT):
            wait_wb(s)

    return emb_lookup


def kernel(x, emb):
    b, h = x.shape
    v, d = emb.shape
    return _build(b, v)(x.astype(jnp.int32), emb)
